# baseline (speedup 1.0000x reference)
import functools

import jax
import jax.numpy as jnp
from jax import lax
from jax.experimental import pallas as pl
from jax.experimental.pallas import tpu as pltpu
from jax.experimental.pallas import tpu_sc as plsc

N, D, H, E = 32768, 768, 128, 64
BN = 1024
BNC = 4096
NC, NS, L = 2, 16, 16
NW = NC * NS
CH = 1
NCK = N // CH
TPW = NCK // NW
SUB = 4
SUBW = TPW // SUB


def _logits_t_body(x_ref, w1_ref, b1_ref, w2_ref, b2_ref, out_ref):
    h = jnp.dot(x_ref[...], w1_ref[...], preferred_element_type=jnp.float32)
    h = jnp.maximum(h + b1_ref[...], 0.0)
    logits_t = lax.dot_general(w2_ref[...], h, (((0,), (1,)), ((), ())),
                               preferred_element_type=jnp.float32)
    out_ref[...] = logits_t + b2_ref[...]


def _logits_t(c, x, W1, b1, W2, b2):
    off = c * (NCK // BN)
    return pl.pallas_call(
        _logits_t_body,
        grid=(NCK // BN,),
        in_specs=[
            pl.BlockSpec((BN, D), lambda i: (off + i, 0)),
            pl.BlockSpec((D, H), lambda i: (0, 0)),
            pl.BlockSpec((1, H), lambda i: (0, 0)),
            pl.BlockSpec((H, E), lambda i: (0, 0)),
            pl.BlockSpec((E, 1), lambda i: (0, 0)),
        ],
        out_specs=pl.BlockSpec((E, BN), lambda i: (0, i)),
        out_shape=jax.ShapeDtypeStruct((E, NCK), jnp.float32),
    )(x, W1, b1.reshape(1, H), W2, b2.reshape(E, 1))


_SC_MESH = plsc.VectorSubcoreMesh(core_axis_name="c", subcore_axis_name="s")


@functools.partial(
    pl.kernel,
    mesh=_SC_MESH,
    out_type=jax.ShapeDtypeStruct((8, NCK), jnp.float32),
    scratch_types=[
        pltpu.VMEM((E, SUBW), jnp.float32),
        pltpu.VMEM((E, SUBW), jnp.float32),
        pltpu.VMEM((5, TPW), jnp.float32),
        pltpu.SemaphoreType.DMA,
        pltpu.SemaphoreType.DMA,
    ],
)
def _sc_gate(logT, out8, buf0, buf1, obuf, sem0, sem1):
    wid = lax.axis_index("s") * NC + lax.axis_index("c")
    base = wid * TPW
    bufs, sems = (buf0, buf1), (sem0, sem1)

    def make_group(buf, off):
        def group(g, _):
            t0 = g * L
            m1 = buf[0, pl.ds(t0, L)]
            i1 = jnp.zeros((L,), jnp.float32)
            m2 = jnp.full((L,), -jnp.inf, jnp.float32)
            i2 = jnp.full((L,), float(E), jnp.float32)
            for e in range(1, E):
                v = buf[e, pl.ds(t0, L)]
                ef = jnp.full((L,), float(e), jnp.float32)
                gt1 = v > m1
                gt2 = v > m2
                m2, i2 = (jnp.where(gt1, m1, jnp.where(gt2, v, m2)),
                          jnp.where(gt1, i1, jnp.where(gt2, ef, i2)))
                m1, i1 = jnp.where(gt1, v, m1), jnp.where(gt1, ef, i1)
            m = jnp.maximum(m1, 0.0)
            e1 = jnp.exp(m1 - m)
            e2 = jnp.exp(m2 - m)
            zv = jnp.exp(0.0 - m)
            rden = 1.0 / (e1 + e2 + (E - 2) * zv)
            vals = (e1 * rden, e2 * rden, zv * rden, i1, i2)
            for k, val in enumerate(vals):
                obuf[k, pl.ds(off + t0, L)] = val
            return 0
        return group

    cps = [None, None]
    cps[0] = pltpu.async_copy(logT.at[:, pl.ds(base, SUBW)], buf0, sem0)
    for s in range(SUB):
        if s + 1 < SUB:
            nb = (s + 1) % 2
            cps[nb] = pltpu.async_copy(
                logT.at[:, pl.ds(base + (s + 1) * SUBW, SUBW)],
                bufs[nb], sems[nb])
        cps[s % 2].wait()
        lax.fori_loop(0, SUBW // L, make_group(bufs[s % 2], s * SUBW), 0)

    for k in range(5):
        pltpu.sync_copy(obuf.at[pl.ds(k, 1)],
                        out8.at[pl.ds(k, 1), pl.ds(base, TPW)])


def _expand_body(c_ref, out_ref):
    c = c_ref[...]

    def bcast(k):
        sk = (lax.broadcasted_iota(jnp.int32, (8, E), 0) == k)
        return lax.dot_general(c, sk.astype(jnp.float32),
                               (((0,), (0,)), ((), ())),
                               preferred_element_type=jnp.float32)

    bw1, bw2, bzv, bi1, bi2 = (bcast(k) for k in range(5))
    colf = lax.broadcasted_iota(jnp.int32, (BNC, E), 1).astype(jnp.float32)
    out_ref[...] = jnp.where(colf == bi1, bw1,
                             jnp.where(colf == bi2, bw2, bzv))


def _expand(c):
    return pl.pallas_call(
        _expand_body,
        grid=(N // BNC,),
        in_specs=[pl.BlockSpec((8, BNC), lambda i: (0, i))],
        out_specs=pl.BlockSpec((BNC, E), lambda i: (i, 0)),
        out_shape=jax.ShapeDtypeStruct((N, E), jnp.float32),
    )(c)


@jax.jit
def kernel(x, W1, b1, W2, b2):
    compact = [_sc_gate(_logits_t(c, x, W1, b1, W2, b2))
               for c in range(CH)]
    return _expand(jnp.concatenate(compact, axis=1))


# speedup vs baseline: 1.6752x; 1.6752x over previous
import functools

import jax
import jax.numpy as jnp
from jax.experimental import pallas as pl

N, D, H, E = 32768, 768, 128, 64
BN = 2048


def _fused_body(x_ref, w1_ref, b1_ref, w2_ref, b2_ref, out_ref):
    h = jnp.dot(x_ref[...], w1_ref[...],
                preferred_element_type=jnp.float32)
    h = jnp.maximum(h + b1_ref[...], 0.0)
    logits = jnp.dot(h, w2_ref[...],
                     preferred_element_type=jnp.float32) + b2_ref[...]
    col = jax.lax.broadcasted_iota(jnp.int32, logits.shape, 1)
    m1 = jnp.max(logits, axis=1, keepdims=True)
    i1 = jnp.min(jnp.where(logits == m1, col, E), axis=1, keepdims=True)
    rest = jnp.where(col == i1, -jnp.inf, logits)
    m2 = jnp.max(rest, axis=1, keepdims=True)
    i2 = jnp.min(jnp.where(rest == m2, col, E), axis=1, keepdims=True)
    m = jnp.maximum(m1, 0.0)
    e1 = jnp.exp(m1 - m)
    e2 = jnp.exp(m2 - m)
    zv = jnp.exp(-m)
    denom = e1 + e2 + (E - 2) * zv
    out_ref[...] = jnp.where(col == i1, e1,
                             jnp.where(col == i2, e2, zv)) / denom


@jax.jit
def kernel(x, W1, b1, W2, b2):
    grid = (N // BN,)
    return pl.pallas_call(
        _fused_body,
        grid=grid,
        in_specs=[
            pl.BlockSpec((BN, D), lambda i: (i, 0)),
            pl.BlockSpec((D, H), lambda i: (0, 0)),
            pl.BlockSpec((1, H), lambda i: (0, 0)),
            pl.BlockSpec((H, E), lambda i: (0, 0)),
            pl.BlockSpec((1, E), lambda i: (0, 0)),
        ],
        out_specs=pl.BlockSpec((BN, E), lambda i: (i, 0)),
        out_shape=jax.ShapeDtypeStruct((N, E), jnp.float32),
    )(x, W1, b1.reshape(1, H), W2, b2.reshape(1, E))


# speedup vs baseline: 1.7592x; 1.0502x over previous
import functools

import jax
import jax.numpy as jnp
from jax.experimental import pallas as pl

N, D, H, E = 32768, 768, 128, 64
BN = 4096


def _fused_body(x_ref, w1_ref, b1_ref, w2_ref, b2_ref, out_ref):
    h = jnp.dot(x_ref[...], w1_ref[...],
                preferred_element_type=jnp.float32)
    h = jnp.maximum(h + b1_ref[...], 0.0)
    logits = jnp.dot(h, w2_ref[...],
                     preferred_element_type=jnp.float32) + b2_ref[...]
    col = jax.lax.broadcasted_iota(jnp.int32, logits.shape, 1)
    m1 = jnp.max(logits, axis=1, keepdims=True)
    i1 = jnp.min(jnp.where(logits == m1, col, E), axis=1, keepdims=True)
    rest = jnp.where(col == i1, -jnp.inf, logits)
    m2 = jnp.max(rest, axis=1, keepdims=True)
    i2 = jnp.min(jnp.where(rest == m2, col, E), axis=1, keepdims=True)
    m = jnp.maximum(m1, 0.0)
    e1 = jnp.exp(m1 - m)
    e2 = jnp.exp(m2 - m)
    zv = jnp.exp(-m)
    denom = e1 + e2 + (E - 2) * zv
    out_ref[...] = jnp.where(col == i1, e1,
                             jnp.where(col == i2, e2, zv)) / denom


@jax.jit
def kernel(x, W1, b1, W2, b2):
    grid = (N // BN,)
    return pl.pallas_call(
        _fused_body,
        grid=grid,
        in_specs=[
            pl.BlockSpec((BN, D), lambda i: (i, 0)),
            pl.BlockSpec((D, H), lambda i: (0, 0)),
            pl.BlockSpec((1, H), lambda i: (0, 0)),
            pl.BlockSpec((H, E), lambda i: (0, 0)),
            pl.BlockSpec((1, E), lambda i: (0, 0)),
        ],
        out_specs=pl.BlockSpec((BN, E), lambda i: (i, 0)),
        out_shape=jax.ShapeDtypeStruct((N, E), jnp.float32),
    )(x, W1, b1.reshape(1, H), W2, b2.reshape(1, E))
